# baseline (speedup 1.0000x reference)
import functools

import jax
import jax.numpy as jnp
from jax import lax
from jax.experimental import pallas as pl
from jax.experimental.pallas import tpu as pltpu
from jax.experimental.pallas import tpu_sc as plsc

NC = 2
NS = 16
NW = NC * NS
CH = 128
BLK = 16


@functools.lru_cache(maxsize=None)
def _make_seg_sum(n, d, e):
    per_w = e // NW
    nch = -(-per_w // (CH * BLK)) * BLK
    per_w_pad = nch * CH
    nblk = nch // BLK
    rows_per_tile = -(-(n + 1) // (NS * 8)) * 8
    n_pad = rows_per_tile * NS

    mesh = plsc.VectorSubcoreMesh(core_axis_name="c", subcore_axis_name="s")

    @functools.partial(
        pl.kernel,
        out_type=jax.ShapeDtypeStruct((NC, n_pad, d), jnp.float32),
        mesh=mesh,
        scratch_types=[
            pltpu.VMEM((BLK, CH), jnp.int32),
            pltpu.VMEM((BLK, CH), jnp.int32),
            pltpu.VMEM((BLK, CH), jnp.int32),
            pltpu.VMEM((BLK, CH), jnp.int32),
            pltpu.VMEM((CH, d), jnp.float32),
            pltpu.VMEM((CH, d), jnp.float32),
            pltpu.VMEM_SHARED((n_pad, d), jnp.float32),
            pltpu.SemaphoreType.DMA,
            pltpu.SemaphoreType.DMA,
            pltpu.SemaphoreType.DMA,
        ],
    )
    def seg_sum(x_hbm, src_hbm, dst_hbm, zeros_hbm, out_hbm,
                sblk_a, sblk_b, dblk_a, dblk_b, rows_a, rows_b, agg,
                sem_a, sem_b, sem_idx):
        c = lax.axis_index("c")
        s = lax.axis_index("s")
        wid = s * NC + c
        base_n = s * rows_per_tile

        def fire(idx, buf, sem):
            pltpu.async_copy(x_hbm.at[idx], buf, sem)

        def drain(idx, dsts, buf, sem):
            pltpu.make_async_copy(x_hbm.at[idx], buf, sem).wait()
            pltpu.sync_copy(buf, agg.at[dsts], add=True)

        pltpu.sync_copy(zeros_hbm.at[pl.ds(base_n, rows_per_tile)],
                        agg.at[pl.ds(base_n, rows_per_tile)])
        pltpu.sync_copy(src_hbm.at[wid].at[pl.ds(0, BLK)], sblk_a)
        pltpu.sync_copy(dst_hbm.at[wid].at[pl.ds(0, BLK)], dblk_a)
        plsc.subcore_barrier()

        sblks, dblks = [sblk_a, sblk_b], [dblk_a, dblk_b]
        for b in range(nblk):
            cs, cd = sblks[b % 2], dblks[b % 2]
            ns_, nd_ = sblks[(b + 1) % 2], dblks[(b + 1) % 2]
            if b + 1 < nblk:
                pltpu.async_copy(src_hbm.at[wid].at[pl.ds((b + 1) * BLK, BLK)],
                                 ns_, sem_idx)
                pltpu.async_copy(dst_hbm.at[wid].at[pl.ds((b + 1) * BLK, BLK)],
                                 nd_, sem_idx)

            @pl.loop(0, BLK)
            def _(k, cs=cs, cd=cd):
                pltpu.async_copy(x_hbm.at[cs.at[k]], rows_a, sem_a).wait()
                pltpu.sync_copy(rows_a, agg.at[cd.at[k]], add=True)

            if b + 1 < nblk:
                pltpu.make_async_copy(
                    src_hbm.at[wid].at[pl.ds(0, BLK)], ns_, sem_idx).wait()
                pltpu.make_async_copy(
                    dst_hbm.at[wid].at[pl.ds(0, BLK)], nd_, sem_idx).wait()

        plsc.subcore_barrier()
        pltpu.sync_copy(agg.at[pl.ds(base_n, rows_per_tile)],
                        out_hbm.at[c].at[pl.ds(base_n, rows_per_tile)])

    return seg_sum, nch, per_w, per_w_pad, n_pad


@functools.lru_cache(maxsize=None)
def _make_dense(n, d_in, d_out, relu):
    blk = 1000
    grid = (n // blk,)

    def body(a0_ref, a1_ref, x_ref, wr_ref, wo_ref, b_ref, o_ref):
        a = a0_ref[...] + a1_ref[...]
        acc = jnp.dot(a, wr_ref[...], preferred_element_type=jnp.float32)
        acc = acc + jnp.dot(x_ref[...], wo_ref[...],
                            preferred_element_type=jnp.float32)
        acc = acc + b_ref[...]
        if relu:
            acc = jnp.maximum(acc, 0.0)
        o_ref[...] = acc

    return pl.pallas_call(
        body,
        grid=grid,
        in_specs=[
            pl.BlockSpec((blk, d_in), lambda i: (i, 0)),
            pl.BlockSpec((blk, d_in), lambda i: (i, 0)),
            pl.BlockSpec((blk, d_in), lambda i: (i, 0)),
            pl.BlockSpec((d_in, d_out), lambda i: (0, 0)),
            pl.BlockSpec((d_in, d_out), lambda i: (0, 0)),
            pl.BlockSpec((1, d_out), lambda i: (0, 0)),
        ],
        out_specs=pl.BlockSpec((blk, d_out), lambda i: (i, 0)),
        out_shape=jax.ShapeDtypeStruct((n, d_out), jnp.float32),
    )


def kernel(x, edge_index, W1_rel, b1, W1_root, W2_rel, b2, W2_root):
    n, d = x.shape
    e = edge_index.shape[1]
    seg_sum, nch, per_w, per_w_pad, n_pad = _make_seg_sum(n, d, e)
    pad = per_w_pad - per_w
    src = jnp.pad(edge_index[0].reshape(NW, per_w),
                  ((0, 0), (0, pad))).reshape(NW, nch, CH)
    dst = jnp.pad(edge_index[1].reshape(NW, per_w), ((0, 0), (0, pad)),
                  constant_values=n).reshape(NW, nch, CH)
    zeros = jnp.zeros((n_pad, d), jnp.float32)

    p1 = seg_sum(x, src, dst, zeros)
    h = _make_dense(n, d, W1_rel.shape[0], True)(
        p1[0], p1[1], x, W1_rel.T, W1_root.T, b1[None, :])
    p2 = seg_sum(h, src, dst, zeros)
    out = _make_dense(n, d, W2_rel.shape[0], False)(
        p2[0], p2[1], h, W2_rel.T, W2_root.T, b2[None, :])
    return out


# speedup vs baseline: 1.0978x; 1.0978x over previous
import functools

import jax
import jax.numpy as jnp
from jax import lax
from jax.experimental import pallas as pl
from jax.experimental.pallas import tpu as pltpu
from jax.experimental.pallas import tpu_sc as plsc

NC = 2
NS = 16
NW = NC * NS
CH = 80
BLK = 16


@functools.lru_cache(maxsize=None)
def _make_seg_sum(n, d, e):
    per_w = e // NW
    nch = -(-per_w // (CH * BLK)) * BLK
    per_w_pad = nch * CH
    nblk = nch // BLK
    rows_per_tile = -(-(n + 1) // (NS * 8)) * 8
    n_pad = rows_per_tile * NS

    mesh = plsc.VectorSubcoreMesh(core_axis_name="c", subcore_axis_name="s")

    @functools.partial(
        pl.kernel,
        out_type=jax.ShapeDtypeStruct((NC, n_pad, d), jnp.float32),
        mesh=mesh,
        scratch_types=[
            pltpu.VMEM((BLK, CH), jnp.int32),
            pltpu.VMEM((BLK, CH), jnp.int32),
            pltpu.VMEM((BLK, CH), jnp.int32),
            pltpu.VMEM((BLK, CH), jnp.int32),
            pltpu.VMEM((CH, d), jnp.float32),
            pltpu.VMEM((CH, d), jnp.float32),
            pltpu.VMEM_SHARED((n_pad, d), jnp.float32),
            pltpu.SemaphoreType.DMA,
            pltpu.SemaphoreType.DMA,
            pltpu.SemaphoreType.DMA,
        ],
    )
    def seg_sum(x_hbm, src_hbm, dst_hbm, zeros_hbm, out_hbm,
                sblk_a, sblk_b, dblk_a, dblk_b, rows_a, rows_b, agg,
                sem_a, sem_b, sem_idx):
        c = lax.axis_index("c")
        s = lax.axis_index("s")
        wid = s * NC + c
        base_n = s * rows_per_tile

        def fire(idx, buf, sem):
            pltpu.async_copy(x_hbm.at[idx], buf, sem)

        def drain(idx, dsts, buf, sem):
            pltpu.make_async_copy(x_hbm.at[idx], buf, sem).wait()
            pltpu.sync_copy(buf, agg.at[dsts], add=True)

        pltpu.sync_copy(zeros_hbm.at[pl.ds(base_n, rows_per_tile)],
                        agg.at[pl.ds(base_n, rows_per_tile)])
        pltpu.sync_copy(src_hbm.at[wid].at[pl.ds(0, BLK)], sblk_a)
        pltpu.sync_copy(dst_hbm.at[wid].at[pl.ds(0, BLK)], dblk_a)
        plsc.subcore_barrier()

        fire(sblk_a.at[0], rows_a, sem_a)
        sblks, dblks = [sblk_a, sblk_b], [dblk_a, dblk_b]
        for b in range(nblk):
            cs, cd = sblks[b % 2], dblks[b % 2]
            ns_, nd_ = sblks[(b + 1) % 2], dblks[(b + 1) % 2]
            if b + 1 < nblk:
                pltpu.async_copy(src_hbm.at[wid].at[pl.ds((b + 1) * BLK, BLK)],
                                 ns_, sem_idx)
                pltpu.async_copy(dst_hbm.at[wid].at[pl.ds((b + 1) * BLK, BLK)],
                                 nd_, sem_idx)

            @pl.loop(0, BLK, step=2)
            def _(k, cs=cs, cd=cd):
                fire(cs.at[k + 1], rows_b, sem_b)
                drain(cs.at[k], cd.at[k], rows_a, sem_a)

                @pl.when(k + 2 < BLK)
                def _():
                    fire(cs.at[k + 2], rows_a, sem_a)

                drain(cs.at[k + 1], cd.at[k + 1], rows_b, sem_b)

            if b + 1 < nblk:
                pltpu.make_async_copy(
                    src_hbm.at[wid].at[pl.ds(0, BLK)], ns_, sem_idx).wait()
                pltpu.make_async_copy(
                    dst_hbm.at[wid].at[pl.ds(0, BLK)], nd_, sem_idx).wait()
                fire(ns_.at[0], rows_a, sem_a)

        plsc.subcore_barrier()
        pltpu.sync_copy(agg.at[pl.ds(base_n, rows_per_tile)],
                        out_hbm.at[c].at[pl.ds(base_n, rows_per_tile)])

    return seg_sum, nch, per_w, per_w_pad, n_pad


@functools.lru_cache(maxsize=None)
def _make_dense(n, d_in, d_out, relu):
    blk = 1000
    grid = (n // blk,)

    def body(a0_ref, a1_ref, x_ref, wr_ref, wo_ref, b_ref, o_ref):
        a = a0_ref[...] + a1_ref[...]
        acc = jnp.dot(a, wr_ref[...], preferred_element_type=jnp.float32)
        acc = acc + jnp.dot(x_ref[...], wo_ref[...],
                            preferred_element_type=jnp.float32)
        acc = acc + b_ref[...]
        if relu:
            acc = jnp.maximum(acc, 0.0)
        o_ref[...] = acc

    return pl.pallas_call(
        body,
        grid=grid,
        in_specs=[
            pl.BlockSpec((blk, d_in), lambda i: (i, 0)),
            pl.BlockSpec((blk, d_in), lambda i: (i, 0)),
            pl.BlockSpec((blk, d_in), lambda i: (i, 0)),
            pl.BlockSpec((d_in, d_out), lambda i: (0, 0)),
            pl.BlockSpec((d_in, d_out), lambda i: (0, 0)),
            pl.BlockSpec((1, d_out), lambda i: (0, 0)),
        ],
        out_specs=pl.BlockSpec((blk, d_out), lambda i: (i, 0)),
        out_shape=jax.ShapeDtypeStruct((n, d_out), jnp.float32),
    )


def kernel(x, edge_index, W1_rel, b1, W1_root, W2_rel, b2, W2_root):
    n, d = x.shape
    e = edge_index.shape[1]
    seg_sum, nch, per_w, per_w_pad, n_pad = _make_seg_sum(n, d, e)
    pad = per_w_pad - per_w
    src = jnp.pad(edge_index[0].reshape(NW, per_w),
                  ((0, 0), (0, pad))).reshape(NW, nch, CH)
    dst = jnp.pad(edge_index[1].reshape(NW, per_w), ((0, 0), (0, pad)),
                  constant_values=n).reshape(NW, nch, CH)
    zeros = jnp.zeros((n_pad, d), jnp.float32)

    p1 = seg_sum(x, src, dst, zeros)
    h = _make_dense(n, d, W1_rel.shape[0], True)(
        p1[0], p1[1], x, W1_rel.T, W1_root.T, b1[None, :])
    p2 = seg_sum(h, src, dst, zeros)
    out = _make_dense(n, d, W2_rel.shape[0], False)(
        p2[0], p2[1], h, W2_rel.T, W2_root.T, b2[None, :])
    return out


# speedup vs baseline: 1.1310x; 1.0302x over previous
import functools

import jax
import jax.numpy as jnp
from jax import lax
from jax.experimental import pallas as pl
from jax.experimental.pallas import tpu as pltpu
from jax.experimental.pallas import tpu_sc as plsc

NC = 2
NS = 16
NW = NC * NS
CH = 128
BLK = 16


@functools.lru_cache(maxsize=None)
def _make_seg_sum(n, d, e):
    per_w = e // NW
    nch = -(-per_w // (CH * BLK)) * BLK
    per_w_pad = nch * CH
    nblk = nch // BLK
    rows_per_tile = -(-(n + NW) // (NS * 8)) * 8
    n_pad = rows_per_tile * NS

    mesh = plsc.VectorSubcoreMesh(core_axis_name="c", subcore_axis_name="s")

    @functools.partial(
        pl.kernel,
        out_type=jax.ShapeDtypeStruct((NC, n_pad, d), jnp.float32),
        mesh=mesh,
        scratch_types=[
            pltpu.VMEM((BLK, CH), jnp.int32),
            pltpu.VMEM((BLK, CH), jnp.int32),
            pltpu.VMEM((BLK, CH), jnp.int32),
            pltpu.VMEM((BLK, CH), jnp.int32),
            pltpu.VMEM((CH, d), jnp.float32),
            pltpu.VMEM((CH, d), jnp.float32),
            pltpu.VMEM_SHARED((n_pad, d), jnp.float32),
            pltpu.SemaphoreType.DMA,
            pltpu.SemaphoreType.DMA,
            pltpu.SemaphoreType.DMA,
            pltpu.SemaphoreType.DMA,
        ],
    )
    def seg_sum(x_hbm, src_hbm, dst_hbm, zeros_hbm, out_hbm,
                sblk_a, sblk_b, dblk_a, dblk_b, rows_a, rows_b, agg,
                sem_a, sem_b, sem_si, sem_di):
        c = lax.axis_index("c")
        s = lax.axis_index("s")
        wid = s * NC + c
        base_n = s * rows_per_tile

        def fire(idx, buf, sem):
            pltpu.async_copy(x_hbm.at[idx], buf, sem)

        def drain(idx, dsts, buf, sem):
            pltpu.make_async_copy(x_hbm.at[idx], buf, sem).wait()
            pltpu.sync_copy(buf, agg.at[dsts], add=True)

        pltpu.sync_copy(zeros_hbm.at[pl.ds(base_n, rows_per_tile)],
                        agg.at[pl.ds(base_n, rows_per_tile)])
        pltpu.sync_copy(src_hbm.at[wid].at[pl.ds(0, BLK)], sblk_a)
        pltpu.sync_copy(dst_hbm.at[wid].at[pl.ds(0, BLK)], dblk_a)
        plsc.subcore_barrier()

        fire(sblk_a.at[0], rows_a, sem_a)
        sblks, dblks = [sblk_a, sblk_b], [dblk_a, dblk_b]
        for b in range(nblk):
            cs, cd = sblks[b % 2], dblks[b % 2]
            ns_, nd_ = sblks[(b + 1) % 2], dblks[(b + 1) % 2]
            if b + 1 < nblk:
                pltpu.async_copy(src_hbm.at[wid].at[pl.ds((b + 1) * BLK, BLK)],
                                 ns_, sem_si)
                pltpu.async_copy(dst_hbm.at[wid].at[pl.ds((b + 1) * BLK, BLK)],
                                 nd_, sem_di)

            @pl.loop(0, BLK, step=2)
            def _(k, cs=cs, cd=cd):
                fire(cs.at[k + 1], rows_b, sem_b)
                drain(cs.at[k], cd.at[k], rows_a, sem_a)

                @pl.when(k + 2 < BLK)
                def _():
                    fire(cs.at[k + 2], rows_a, sem_a)

                drain(cs.at[k + 1], cd.at[k + 1], rows_b, sem_b)

            if b + 1 < nblk:
                pltpu.make_async_copy(
                    src_hbm.at[wid].at[pl.ds(0, BLK)], ns_, sem_si).wait()
                pltpu.make_async_copy(
                    dst_hbm.at[wid].at[pl.ds(0, BLK)], nd_, sem_di).wait()
                fire(ns_.at[0], rows_a, sem_a)

        plsc.subcore_barrier()
        pltpu.sync_copy(agg.at[pl.ds(base_n, rows_per_tile)],
                        out_hbm.at[c].at[pl.ds(base_n, rows_per_tile)])

    return seg_sum, nch, per_w, per_w_pad, n_pad


@functools.lru_cache(maxsize=None)
def _make_dense(n, d_in, d_out, relu):
    blk = 1000
    grid = (n // blk,)

    def body(a0_ref, a1_ref, x_ref, wr_ref, wo_ref, b_ref, o_ref):
        a = a0_ref[...] + a1_ref[...]
        acc = jnp.dot(a, wr_ref[...], preferred_element_type=jnp.float32)
        acc = acc + jnp.dot(x_ref[...], wo_ref[...],
                            preferred_element_type=jnp.float32)
        acc = acc + b_ref[...]
        if relu:
            acc = jnp.maximum(acc, 0.0)
        o_ref[...] = acc

    return pl.pallas_call(
        body,
        grid=grid,
        in_specs=[
            pl.BlockSpec((blk, d_in), lambda i: (i, 0)),
            pl.BlockSpec((blk, d_in), lambda i: (i, 0)),
            pl.BlockSpec((blk, d_in), lambda i: (i, 0)),
            pl.BlockSpec((d_in, d_out), lambda i: (0, 0)),
            pl.BlockSpec((d_in, d_out), lambda i: (0, 0)),
            pl.BlockSpec((1, d_out), lambda i: (0, 0)),
        ],
        out_specs=pl.BlockSpec((blk, d_out), lambda i: (i, 0)),
        out_shape=jax.ShapeDtypeStruct((n, d_out), jnp.float32),
    )


def kernel(x, edge_index, W1_rel, b1, W1_root, W2_rel, b2, W2_root):
    n, d = x.shape
    e = edge_index.shape[1]
    seg_sum, nch, per_w, per_w_pad, n_pad = _make_seg_sum(n, d, e)
    pad = per_w_pad - per_w
    src = jnp.pad(edge_index[0].reshape(NW, per_w),
                  ((0, 0), (0, pad))).reshape(NW, nch, CH)
    dump = jnp.broadcast_to(n + jnp.arange(NW, dtype=jnp.int32)[:, None],
                            (NW, pad))
    dst = jnp.concatenate([edge_index[1].reshape(NW, per_w), dump],
                          axis=1).reshape(NW, nch, CH)
    zeros = jnp.zeros((n_pad, d), jnp.float32)

    p1 = seg_sum(x, src, dst, zeros)
    h = _make_dense(n, d, W1_rel.shape[0], True)(
        p1[0], p1[1], x, W1_rel.T, W1_root.T, b1[None, :])
    p2 = seg_sum(h, src, dst, zeros)
    out = _make_dense(n, d, W2_rel.shape[0], False)(
        p2[0], p2[1], h, W2_rel.T, W2_root.T, b2[None, :])
    return out


# speedup vs baseline: 3.1770x; 2.8091x over previous
import functools

import jax
import jax.numpy as jnp
from jax import lax
from jax.experimental import pallas as pl
from jax.experimental.pallas import tpu as pltpu
from jax.experimental.pallas import tpu_sc as plsc

NC = 2
NS = 16
NW = NC * NS
CH = 80


@functools.lru_cache(maxsize=None)
def _make_seg_sum(n, d, e):
    per_w = e // NW
    assert per_w % CH == 0
    nch = per_w // CH
    rows_per_tile = -(-n // (NS * 8)) * 8
    n_pad = rows_per_tile * NS

    mesh = plsc.VectorSubcoreMesh(core_axis_name="c", subcore_axis_name="s")

    @functools.partial(
        pl.kernel,
        out_type=jax.ShapeDtypeStruct((NC, n_pad, d), jnp.float32),
        mesh=mesh,
        scratch_types=[
            pltpu.VMEM((per_w,), jnp.int32),
            pltpu.VMEM((nch, CH), jnp.int32),
            pltpu.VMEM((CH, d), jnp.float32),
            pltpu.VMEM((CH, d), jnp.float32),
            pltpu.VMEM_SHARED((n_pad, d), jnp.float32),
            pltpu.SemaphoreType.DMA,
            pltpu.SemaphoreType.DMA,
        ],
    )
    def seg_sum(x_hbm, src_hbm, dst_hbm, zeros_hbm, out_hbm,
                srcb, dstb, rows_a, rows_b, agg, sem_a, sem_b):
        c = lax.axis_index("c")
        s = lax.axis_index("s")
        wid = s * NC + c
        base_n = s * rows_per_tile

        def fire(j, buf, sem):
            pltpu.async_copy(x_hbm.at[srcb.at[pl.ds(j * CH, CH)]], buf, sem)

        def drain(j, buf, sem):
            pltpu.make_async_copy(x_hbm.at[srcb.at[pl.ds(j * CH, CH)]],
                                  buf, sem).wait()
            pltpu.sync_copy(buf, agg.at[dstb.at[j]], add=True)

        pltpu.sync_copy(zeros_hbm.at[pl.ds(base_n, rows_per_tile)],
                        agg.at[pl.ds(base_n, rows_per_tile)])
        pltpu.sync_copy(src_hbm.at[wid], srcb)
        pltpu.sync_copy(dst_hbm.at[wid], dstb)
        plsc.subcore_barrier()

        assert nch % 2 == 1
        fire(0, rows_a, sem_a)

        @pl.loop(0, nch - 1, step=2)
        def _(j):
            fire(j + 1, rows_b, sem_b)
            drain(j, rows_a, sem_a)
            fire(j + 2, rows_a, sem_a)
            drain(j + 1, rows_b, sem_b)

        drain(nch - 1, rows_a, sem_a)

        plsc.subcore_barrier()
        pltpu.sync_copy(agg.at[pl.ds(base_n, rows_per_tile)],
                        out_hbm.at[c].at[pl.ds(base_n, rows_per_tile)])

    return seg_sum, nch, per_w, n_pad


@functools.lru_cache(maxsize=None)
def _make_dense(n, d_in, d_out, relu):
    blk = 1000
    grid = (n // blk,)

    def body(a0_ref, a1_ref, x_ref, wr_ref, wo_ref, b_ref, o_ref):
        a = a0_ref[...] + a1_ref[...]
        acc = jnp.dot(a, wr_ref[...], preferred_element_type=jnp.float32)
        acc = acc + jnp.dot(x_ref[...], wo_ref[...],
                            preferred_element_type=jnp.float32)
        acc = acc + b_ref[...]
        if relu:
            acc = jnp.maximum(acc, 0.0)
        o_ref[...] = acc

    return pl.pallas_call(
        body,
        grid=grid,
        in_specs=[
            pl.BlockSpec((blk, d_in), lambda i: (i, 0)),
            pl.BlockSpec((blk, d_in), lambda i: (i, 0)),
            pl.BlockSpec((blk, d_in), lambda i: (i, 0)),
            pl.BlockSpec((d_in, d_out), lambda i: (0, 0)),
            pl.BlockSpec((d_in, d_out), lambda i: (0, 0)),
            pl.BlockSpec((1, d_out), lambda i: (0, 0)),
        ],
        out_specs=pl.BlockSpec((blk, d_out), lambda i: (i, 0)),
        out_shape=jax.ShapeDtypeStruct((n, d_out), jnp.float32),
    )


def kernel(x, edge_index, W1_rel, b1, W1_root, W2_rel, b2, W2_root):
    n, d = x.shape
    e = edge_index.shape[1]
    seg_sum, nch, per_w, n_pad = _make_seg_sum(n, d, e)
    src = edge_index[0].reshape(NW, per_w)
    dst = edge_index[1].reshape(NW, nch, CH)
    zeros = jnp.zeros((n_pad, d), jnp.float32)

    p1 = seg_sum(x, src, dst, zeros)
    h = _make_dense(n, d, W1_rel.shape[0], True)(
        p1[0], p1[1], x, W1_rel.T, W1_root.T, b1[None, :])
    p2 = seg_sum(h, src, dst, zeros)
    out = _make_dense(n, d, W2_rel.shape[0], False)(
        p2[0], p2[1], h, W2_rel.T, W2_root.T, b2[None, :])
    return out
